# baseline (speedup 1.0000x reference)
import functools

import jax
import jax.numpy as jnp
from jax import lax
from jax.experimental import pallas as pl
from jax.experimental.pallas import tpu as pltpu
from jax.experimental.pallas import tpu_sc as plsc

_NC = 2
_NS = 16
_CH = 128
_G = 16

_HIGH = jax.lax.Precision.HIGHEST


def _dot(a, b):
    return jax.lax.dot_general(a, b, (((1,), (0,)), ((), ())),
                               precision=_HIGH,
                               preferred_element_type=jnp.float32)


_GI = 4


def _sc_segsum(n_rel, rel_tab, n_tabs, k, np1, fc):
    mesh = plsc.VectorSubcoreMesh(core_axis_name="c", subcore_axis_name="s",
                                  num_cores=_NC, num_subcores=_NS)
    out = jax.ShapeDtypeStruct((_NC, n_rel, np1, fc), jnp.float32)
    zr = np1 // _NS
    scratch = [
        pltpu.VMEM_SHARED((np1, fc), jnp.float32),
        pltpu.VMEM((2, _CH), jnp.int32),
        pltpu.VMEM((_CH, fc), jnp.float32),
        pltpu.SemaphoreType.DMA,
    ]

    first_cnt = min((i for i, t in enumerate(rel_tab) if t < 0),
                    default=None)

    def body(*refs):
        tabs = refs[:n_tabs]
        if first_cnt is not None:
            sds, zf, ones, sums_o = refs[n_tabs:n_tabs + 4]
            sc0 = n_tabs + 4
        else:
            sds, zf, sums_o = refs[n_tabs:n_tabs + 3]
            sc0 = n_tabs + 3
        acc, sd_v, rbuf, sem = refs[sc0:]
        c = lax.axis_index("c")
        s = lax.axis_index("s")
        pltpu.sync_copy(zf.at[pl.ds(s * zr, zr)], acc.at[pl.ds(s * zr, zr)])
        plsc.subcore_barrier()
        for r in range(n_rel):
            if r == first_cnt:
                pltpu.sync_copy(ones, rbuf)
            if rel_tab[r] >= 0:
                tab = tabs[rel_tab[r]]

                @pl.loop(0, k)
                def _(j):
                    pltpu.sync_copy(sds.at[c, r, s, j], sd_v)
                    pltpu.async_copy(tab.at[sd_v.at[0]], rbuf, sem).wait()
                    pltpu.sync_copy(rbuf, acc.at[sd_v.at[1]], add=True)
            else:
                @pl.loop(0, k)
                def _(j):
                    pltpu.sync_copy(sds.at[c, r, s, j], sd_v)
                    pltpu.sync_copy(rbuf, acc.at[sd_v.at[1]], add=True)

            plsc.subcore_barrier()
            pltpu.sync_copy(acc.at[pl.ds(s * zr, zr)],
                            sums_o.at[c, r, pl.ds(s * zr, zr)])
            if r < n_rel - 1:
                pltpu.sync_copy(zf.at[pl.ds(s * zr, zr)],
                                acc.at[pl.ds(s * zr, zr)])
            plsc.subcore_barrier()

    return pl.kernel(body, out_type=out, mesh=mesh, scratch_types=scratch)


def _tc_combine0(n, bn, d, h):
    grid = (n // bn,)

    def body(S, C, xh, xc, xg, Wl, bl, Wr, oh, oc, og):
        m = (S[0] + S[1]) / jnp.maximum(C[0] + C[1], 1.0)
        og[...] = jax.nn.relu(_dot(m[0], Wl[0]) + bl[0] + _dot(xg[...], Wr[0]))
        oh[...] = jax.nn.relu(_dot(m[1], Wl[1]) + bl[1] + _dot(xh[...], Wr[1]))
        oc[...] = jax.nn.relu(
            _dot(m[2], Wl[2]) + _dot(m[3], Wl[3]) + _dot(m[4], Wl[4])
            + (bl[2] + bl[3] + bl[4])
            + _dot(xc[...], Wr[2] + Wr[3] + Wr[4]))

    return pl.pallas_call(
        body,
        grid=grid,
        in_specs=[
            pl.BlockSpec((2, 5, bn, d), lambda i: (0, 0, i, 0)),
            pl.BlockSpec((2, 5, bn, 1), lambda i: (0, 0, i, 0)),
            pl.BlockSpec((bn, d), lambda i: (i, 0)),
            pl.BlockSpec((bn, d), lambda i: (i, 0)),
            pl.BlockSpec((bn, d), lambda i: (i, 0)),
            pl.BlockSpec((5, d, h), lambda i: (0, 0, 0)),
            pl.BlockSpec((5, h), lambda i: (0, 0)),
            pl.BlockSpec((5, d, h), lambda i: (0, 0, 0)),
        ],
        out_specs=[
            pl.BlockSpec((bn, h), lambda i: (i, 0)),
            pl.BlockSpec((bn, h), lambda i: (i, 0)),
            pl.BlockSpec((bn, h), lambda i: (i, 0)),
        ],
        out_shape=[jax.ShapeDtypeStruct((n, h), jnp.float32)] * 3,
    )


def _tc_final(n, bn, h, h2, out_d):
    grid = (n // bn,)
    steps = n // bn

    def body(S, C, xc, Wl, bls, Wrs, b2d, f1w, f1b, f2w, f2b, out, accP, accC):
        i = pl.program_id(0)

        @pl.when(i == 0)
        def _():
            accP[...] = jnp.zeros_like(accP)
            accC[...] = jnp.zeros_like(accC)

        m = S[...] / jnp.maximum(C[0] + C[1], 1.0)
        o = jax.nn.relu(
            _dot(m[0], Wl[0]) + _dot(m[1], Wl[1]) + _dot(m[2], Wl[2])
            + bls[...] + _dot(xc[...], Wrs[...]))
        mask = (lax.broadcasted_iota(jnp.int32, (_G, bn), 0)
                == b2d[...].reshape(1, bn)).astype(jnp.float32)
        accP[...] += _dot(mask, o)
        accC[...] += jnp.broadcast_to(
            jnp.sum(mask, axis=1, keepdims=True), accC.shape)

        @pl.when(i == steps - 1)
        def _():
            pooled = accP[...] / jnp.maximum(accC[...], 1.0)
            hh = jax.nn.relu(_dot(pooled, f1w[...]) + f1b[...])
            out[...] = _dot(hh, f2w[...]) + f2b[...]

    return pl.pallas_call(
        body,
        grid=grid,
        in_specs=[
            pl.BlockSpec((3, bn, h), lambda i: (0, i, 0)),
            pl.BlockSpec((2, 3, bn, 1), lambda i: (0, 0, i, 0)),
            pl.BlockSpec((bn, h), lambda i: (i, 0)),
            pl.BlockSpec((3, h, h), lambda i: (0, 0, 0)),
            pl.BlockSpec((1, h), lambda i: (0, 0)),
            pl.BlockSpec((h, h), lambda i: (0, 0)),
            pl.BlockSpec((1, 1, bn), lambda i: (i, 0, 0)),
            pl.BlockSpec((h, h2), lambda i: (0, 0)),
            pl.BlockSpec((1, h2), lambda i: (0, 0)),
            pl.BlockSpec((h2, out_d), lambda i: (0, 0)),
            pl.BlockSpec((1, out_d), lambda i: (0, 0)),
        ],
        out_specs=pl.BlockSpec((_G, out_d), lambda i: (0, 0)),
        out_shape=jax.ShapeDtypeStruct((_G, out_d), jnp.float32),
        scratch_shapes=[
            pltpu.VMEM((_G, h), jnp.float32),
            pltpu.VMEM((_G, h), jnp.float32),
        ],
    )


def _split_flat(x, fc):
    return jnp.concatenate([x[:, :fc], x[:, fc:]], axis=0)


def kernel(x_hru, x_channel, x_gw_cell, ei_hru_gw, ei_ch_hru, ei_gw_ch,
           ei_hru_ch, ei_ch_ch, batch, Wl0, bl0, Wr0, Wl1, bl1, Wr1,
           fc1_w, fc1_b, fc2_w, fc2_b):
    n, d = x_hru.shape
    h = Wl0.shape[2]
    h2 = fc1_w.shape[1]
    out_d = fc2_w.shape[1]
    eis = [ei_hru_gw, ei_ch_hru, ei_gw_ch, ei_hru_ch, ei_ch_ch]
    e = eis[0].shape[1]

    chunk0 = _NC * _NS * _CH
    k0 = -(-(-(-e // chunk0)) // _GI) * _GI
    pad0 = k0 * chunk0 - e
    chunk1 = _NS * _CH
    k1 = -(-(-(-e // chunk1)) // _GI) * _GI
    pad1 = k1 * chunk1 - e
    src0_l, dst0_l, src1_l, dst1_l = [], [], [], []
    for ei in eis:
        s_ = ei[0].astype(jnp.int32)
        d_ = ei[1].astype(jnp.int32)
        src0_l.append(jnp.concatenate(
            [s_, jnp.zeros((pad0,), jnp.int32)]).reshape(_NC, _NS, k0, _CH))
        dst0_l.append(jnp.concatenate(
            [d_, jnp.full((pad0,), n, jnp.int32)]).reshape(_NC, _NS, k0, _CH))
        src1_l.append(jnp.concatenate(
            [s_, jnp.zeros((pad1,), jnp.int32)]).reshape(_NS, k1, _CH))
        dst1_l.append(jnp.concatenate(
            [d_, jnp.full((pad1,), n, jnp.int32)]).reshape(_NS, k1, _CH))
    srcs0 = jnp.stack(src0_l, axis=1)
    dsts0 = jnp.stack(dst0_l, axis=1)
    srcs0 = jnp.concatenate([srcs0, jnp.zeros_like(srcs0)], axis=1)
    dsts0 = jnp.concatenate([dsts0, dsts0], axis=1)
    sds0 = jnp.stack([srcs0, dsts0], axis=4)
    src1 = jnp.stack(src1_l[2:])
    dst1 = jnp.stack(dst1_l[2:])
    srcs1 = jnp.stack([src1, src1 + n])
    dsts1 = jnp.stack([dst1, dst1])
    sds1 = jnp.stack([srcs1, dsts1], axis=4)

    np1 = -(-(n + 1) // 128) * 128
    fc1 = h // 2
    zf0 = jnp.zeros((np1, d), jnp.float32)
    zf1 = jnp.zeros((np1, fc1), jnp.float32)
    ones0 = jnp.ones((_CH, d), jnp.float32)

    seg0 = _sc_segsum(10, (0, 1, 2, 0, 1, -1, -1, -1, -1, -1), 3, k0, np1, d)
    sums0 = seg0(x_hru, x_channel, x_gw_cell, sds0, zf0, ones0)
    S0 = sums0[:, :5, :n]
    C0 = sums0[:, 5:, :n, 0:1]

    bn = 1000
    xh1, xc1, xg1 = _tc_combine0(n, bn, d, h)(
        S0, C0, x_hru, x_channel, x_gw_cell, Wl0, bl0, Wr0)

    seg1 = _sc_segsum(3, (0, 1, 2), 3, k1, np1, fc1)
    sums1 = seg1(_split_flat(xg1, fc1), _split_flat(xh1, fc1),
                 _split_flat(xc1, fc1), sds1, zf1)
    S1 = jnp.concatenate([sums1[0], sums1[1]], axis=-1)[:, :n]
    C1 = C0[:, 2:5]

    out = _tc_final(n, bn, h, h2, out_d)(
        S1, C1, xc1, Wl1[2:5],
        (bl1[2] + bl1[3] + bl1[4]).reshape(1, h),
        Wr1[2] + Wr1[3] + Wr1[4],
        batch.astype(jnp.int32).reshape(n // bn, 1, bn),
        fc1_w, fc1_b.reshape(1, h2), fc2_w, fc2_b.reshape(1, out_d))
    return out


# speedup vs baseline: 1.4541x; 1.4541x over previous
import functools

import jax
import jax.numpy as jnp
from jax import lax
from jax.experimental import pallas as pl
from jax.experimental.pallas import tpu as pltpu
from jax.experimental.pallas import tpu_sc as plsc

_NC = 2
_NS = 16
_CH = 128
_G = 16

_HIGH = jax.lax.Precision.HIGHEST


def _dot(a, b):
    return jax.lax.dot_general(a, b, (((1,), (0,)), ((), ())),
                               precision=_HIGH,
                               preferred_element_type=jnp.float32)


_GI = 4


def _sc_segsum(n_rel, rel_tab, n_tabs, k, np1, fc):
    mesh = plsc.VectorSubcoreMesh(core_axis_name="c", subcore_axis_name="s",
                                  num_cores=_NC, num_subcores=_NS)
    out = jax.ShapeDtypeStruct((_NC, n_rel, np1, fc), jnp.float32)
    zr = np1 // _NS
    scratch = [
        pltpu.VMEM_SHARED((np1, fc), jnp.float32),
        pltpu.VMEM((2, _CH), jnp.int32),
        pltpu.VMEM((_CH, fc), jnp.float32),
        pltpu.SemaphoreType.DMA,
    ]

    first_cnt = min((i for i, t in enumerate(rel_tab) if t < 0),
                    default=None)

    def body(*refs):
        tabs = refs[:n_tabs]
        if first_cnt is not None:
            sds, zf, ones, sums_o = refs[n_tabs:n_tabs + 4]
            sc0 = n_tabs + 4
        else:
            sds, zf, sums_o = refs[n_tabs:n_tabs + 3]
            sc0 = n_tabs + 3
        acc, sd_v, rbuf, sem = refs[sc0:]
        c = lax.axis_index("c")
        s = lax.axis_index("s")
        pltpu.sync_copy(zf.at[pl.ds(s * zr, zr)], acc.at[pl.ds(s * zr, zr)])
        plsc.subcore_barrier()
        for r in range(n_rel):
            if r == first_cnt:
                pltpu.sync_copy(ones, rbuf)
            if rel_tab[r] >= 0:
                tab = tabs[rel_tab[r]]

                @pl.loop(0, k)
                def _(j):
                    pltpu.sync_copy(sds.at[c, r, s, j], sd_v)
                    pltpu.async_copy(tab.at[sd_v.at[0]], rbuf, sem).wait()
                    pltpu.sync_copy(rbuf, acc.at[sd_v.at[1]], add=True)
            else:
                @pl.loop(0, k)
                def _(j):
                    pltpu.sync_copy(sds.at[c, r, s, j], sd_v)
                    pltpu.sync_copy(rbuf, acc.at[sd_v.at[1]], add=True)

            plsc.subcore_barrier()
            pltpu.sync_copy(acc.at[pl.ds(s * zr, zr)],
                            sums_o.at[c, r, pl.ds(s * zr, zr)])
            if r < n_rel - 1:
                pltpu.sync_copy(zf.at[pl.ds(s * zr, zr)],
                                acc.at[pl.ds(s * zr, zr)])
            plsc.subcore_barrier()

    return pl.kernel(body, out_type=out, mesh=mesh, scratch_types=scratch)


def _tc_combine0(n, bn, d, h):
    grid = (n // bn,)

    def body(S, C, xh, xc, xg, Wl, bl, Wr, oh, oc, og):
        m = (S[0] + S[1]) / jnp.maximum(C[0] + C[1], 1.0)
        og[...] = jax.nn.relu(_dot(m[0], Wl[0]) + bl[0] + _dot(xg[...], Wr[0]))
        oh[...] = jax.nn.relu(_dot(m[1], Wl[1]) + bl[1] + _dot(xh[...], Wr[1]))
        oc[...] = jax.nn.relu(
            _dot(m[2], Wl[2]) + _dot(m[3], Wl[3]) + _dot(m[4], Wl[4])
            + (bl[2] + bl[3] + bl[4])
            + _dot(xc[...], Wr[2] + Wr[3] + Wr[4]))

    return pl.pallas_call(
        body,
        grid=grid,
        in_specs=[
            pl.BlockSpec((2, 5, bn, d), lambda i: (0, 0, i, 0)),
            pl.BlockSpec((2, 5, bn, 1), lambda i: (0, 0, i, 0)),
            pl.BlockSpec((bn, d), lambda i: (i, 0)),
            pl.BlockSpec((bn, d), lambda i: (i, 0)),
            pl.BlockSpec((bn, d), lambda i: (i, 0)),
            pl.BlockSpec((5, d, h), lambda i: (0, 0, 0)),
            pl.BlockSpec((5, h), lambda i: (0, 0)),
            pl.BlockSpec((5, d, h), lambda i: (0, 0, 0)),
        ],
        out_specs=[
            pl.BlockSpec((bn, h), lambda i: (i, 0)),
            pl.BlockSpec((bn, h), lambda i: (i, 0)),
            pl.BlockSpec((bn, h), lambda i: (i, 0)),
        ],
        out_shape=[jax.ShapeDtypeStruct((n, h), jnp.float32)] * 3,
    )


def _tc_final(n, bn, h, h2, out_d):
    grid = (n // bn,)
    steps = n // bn

    def body(S, C, xc, Wl, bls, Wrs, b2d, f1w, f1b, f2w, f2b, out, accP, accC):
        i = pl.program_id(0)

        @pl.when(i == 0)
        def _():
            accP[...] = jnp.zeros_like(accP)
            accC[...] = jnp.zeros_like(accC)

        m = S[...] / jnp.maximum(C[0] + C[1], 1.0)
        o = jax.nn.relu(
            _dot(m[0], Wl[0]) + _dot(m[1], Wl[1]) + _dot(m[2], Wl[2])
            + bls[...] + _dot(xc[...], Wrs[...]))
        mask = (lax.broadcasted_iota(jnp.int32, (_G, bn), 0)
                == b2d[...].reshape(1, bn)).astype(jnp.float32)
        accP[...] += _dot(mask, o)
        accC[...] += jnp.broadcast_to(
            jnp.sum(mask, axis=1, keepdims=True), accC.shape)

        @pl.when(i == steps - 1)
        def _():
            pooled = accP[...] / jnp.maximum(accC[...], 1.0)
            hh = jax.nn.relu(_dot(pooled, f1w[...]) + f1b[...])
            out[...] = _dot(hh, f2w[...]) + f2b[...]

    return pl.pallas_call(
        body,
        grid=grid,
        in_specs=[
            pl.BlockSpec((3, bn, h), lambda i: (0, i, 0)),
            pl.BlockSpec((2, 3, bn, 1), lambda i: (0, 0, i, 0)),
            pl.BlockSpec((bn, h), lambda i: (i, 0)),
            pl.BlockSpec((3, h, h), lambda i: (0, 0, 0)),
            pl.BlockSpec((1, h), lambda i: (0, 0)),
            pl.BlockSpec((h, h), lambda i: (0, 0)),
            pl.BlockSpec((1, 1, bn), lambda i: (i, 0, 0)),
            pl.BlockSpec((h, h2), lambda i: (0, 0)),
            pl.BlockSpec((1, h2), lambda i: (0, 0)),
            pl.BlockSpec((h2, out_d), lambda i: (0, 0)),
            pl.BlockSpec((1, out_d), lambda i: (0, 0)),
        ],
        out_specs=pl.BlockSpec((_G, out_d), lambda i: (0, 0)),
        out_shape=jax.ShapeDtypeStruct((_G, out_d), jnp.float32),
        scratch_shapes=[
            pltpu.VMEM((_G, h), jnp.float32),
            pltpu.VMEM((_G, h), jnp.float32),
        ],
    )


def _split_flat(x, fc):
    return jnp.concatenate([x[:, :fc], x[:, fc:]], axis=0)


def kernel(x_hru, x_channel, x_gw_cell, ei_hru_gw, ei_ch_hru, ei_gw_ch,
           ei_hru_ch, ei_ch_ch, batch, Wl0, bl0, Wr0, Wl1, bl1, Wr1,
           fc1_w, fc1_b, fc2_w, fc2_b):
    n, d = x_hru.shape
    h = Wl0.shape[2]
    h2 = fc1_w.shape[1]
    out_d = fc2_w.shape[1]
    eis = [ei_hru_gw, ei_ch_hru, ei_gw_ch, ei_hru_ch, ei_ch_ch]
    e = eis[0].shape[1]

    chunk0 = _NC * _NS * _CH
    k0 = -(-e // chunk0)
    pad0 = k0 * chunk0 - e
    chunk1 = _NS * _CH
    k1 = -(-e // chunk1)
    pad1 = k1 * chunk1 - e
    src0_l, dst0_l, src1_l, dst1_l = [], [], [], []
    for ei in eis:
        s_ = ei[0].astype(jnp.int32)
        d_ = ei[1].astype(jnp.int32)
        src0_l.append(jnp.concatenate(
            [s_, jnp.zeros((pad0,), jnp.int32)]).reshape(_NC, _NS, k0, _CH))
        dst0_l.append(jnp.concatenate(
            [d_, jnp.full((pad0,), n, jnp.int32)]).reshape(_NC, _NS, k0, _CH))
        src1_l.append(jnp.concatenate(
            [s_, jnp.zeros((pad1,), jnp.int32)]).reshape(_NS, k1, _CH))
        dst1_l.append(jnp.concatenate(
            [d_, jnp.full((pad1,), n, jnp.int32)]).reshape(_NS, k1, _CH))
    srcs0 = jnp.stack(src0_l, axis=1)
    dsts0 = jnp.stack(dst0_l, axis=1)
    srcs0 = jnp.concatenate([srcs0, jnp.zeros_like(srcs0)], axis=1)
    dsts0 = jnp.concatenate([dsts0, dsts0], axis=1)
    sds0 = jnp.stack([srcs0, dsts0], axis=4)
    src1 = jnp.stack(src1_l[2:])
    dst1 = jnp.stack(dst1_l[2:])
    srcs1 = jnp.stack([src1, src1 + n])
    dsts1 = jnp.stack([dst1, dst1])
    sds1 = jnp.stack([srcs1, dsts1], axis=4)

    np1 = -(-(n + 1) // 128) * 128
    fc1 = h // 2
    zf0 = jnp.zeros((np1, d), jnp.float32)
    zf1 = jnp.zeros((np1, fc1), jnp.float32)
    ones0 = jnp.ones((_CH, d), jnp.float32)

    seg0 = _sc_segsum(10, (0, 1, 2, 0, 1, -1, -1, -1, -1, -1), 3, k0, np1, d)
    sums0 = seg0(x_hru, x_channel, x_gw_cell, sds0, zf0, ones0)
    S0 = sums0[:, :5, :n]
    C0 = sums0[:, 5:, :n, 0:1]

    bn = 1000
    xh1, xc1, xg1 = _tc_combine0(n, bn, d, h)(
        S0, C0, x_hru, x_channel, x_gw_cell, Wl0, bl0, Wr0)

    seg1 = _sc_segsum(3, (0, 1, 2), 3, k1, np1, fc1)
    sums1 = seg1(_split_flat(xg1, fc1), _split_flat(xh1, fc1),
                 _split_flat(xc1, fc1), sds1, zf1)
    S1 = jnp.concatenate([sums1[0], sums1[1]], axis=-1)[:, :n]
    C1 = C0[:, 2:5]

    out = _tc_final(n, bn, h, h2, out_d)(
        S1, C1, xc1, Wl1[2:5],
        (bl1[2] + bl1[3] + bl1[4]).reshape(1, h),
        Wr1[2] + Wr1[3] + Wr1[4],
        batch.astype(jnp.int32).reshape(n // bn, 1, bn),
        fc1_w, fc1_b.reshape(1, h2), fc2_w, fc2_b.reshape(1, out_d))
    return out


# speedup vs baseline: 2.5233x; 1.7353x over previous
import functools

import jax
import jax.numpy as jnp
from jax import lax
from jax.experimental import pallas as pl
from jax.experimental.pallas import tpu as pltpu
from jax.experimental.pallas import tpu_sc as plsc

_NC = 2
_NS = 16
_CH = 128
_G = 16

_HIGH = jax.lax.Precision.HIGHEST


def _dot(a, b):
    return jax.lax.dot_general(a, b, (((1,), (0,)), ((), ())),
                               precision=_HIGH,
                               preferred_element_type=jnp.float32)


_GI = 4


def _sc_segsum(n_rel, rel_tab, n_tabs, k, np1, fc):
    mesh = plsc.VectorSubcoreMesh(core_axis_name="c", subcore_axis_name="s",
                                  num_cores=_NC, num_subcores=_NS)
    out = jax.ShapeDtypeStruct((_NC, n_rel, np1, fc), jnp.float32)
    zr = np1 // _NS
    assert k % _GI == 0
    ng = k // _GI
    scratch = [
        pltpu.VMEM_SHARED((np1, fc), jnp.float32),
        pltpu.VMEM((_GI, 2, _CH), jnp.int32),
        pltpu.VMEM((2, _CH, fc), jnp.float32),
        pltpu.SemaphoreType.DMA,
        pltpu.SemaphoreType.DMA,
        pltpu.SemaphoreType.DMA,
        pltpu.SemaphoreType.DMA,
    ]

    first_cnt = min((i for i, t in enumerate(rel_tab) if t < 0),
                    default=None)

    def body(*refs):
        tabs = refs[:n_tabs]
        if first_cnt is not None:
            sds, zf, ones, sums_o = refs[n_tabs:n_tabs + 4]
            sc0 = n_tabs + 4
        else:
            sds, zf, sums_o = refs[n_tabs:n_tabs + 3]
            sc0 = n_tabs + 3
        acc, sdq, rb, gs0, gs1, ss0, ss1 = refs[sc0:]
        c = lax.axis_index("c")
        s = lax.axis_index("s")
        rb0, rb1 = rb.at[0], rb.at[1]

        def _g(tab, rbx, sem, i):
            return pltpu.make_async_copy(tab.at[sdq.at[i, 0]], rbx, sem)

        def _s(rbx, sem, i):
            return pltpu.make_async_copy(rbx, acc.at[sdq.at[i, 1]], sem)

        pltpu.sync_copy(zf.at[pl.ds(s * zr, zr)], acc.at[pl.ds(s * zr, zr)])
        plsc.subcore_barrier()
        for r in range(n_rel):
            if r == first_cnt:
                pltpu.sync_copy(ones, rb0)
            if rel_tab[r] >= 0:
                tab = tabs[rel_tab[r]]

                @pl.loop(0, ng)
                def _(g):
                    pltpu.sync_copy(sds.at[c, r, s, g], sdq)
                    _g(tab, rb0, gs0, 0).start()
                    _g(tab, rb1, gs1, 1).start()
                    _g(tab, rb0, gs0, 0).wait()
                    _s(rb0, ss0, 0).start(add=True)
                    _g(tab, rb1, gs1, 1).wait()
                    _s(rb0, ss0, 0).wait()
                    _g(tab, rb0, gs0, 2).start()
                    _s(rb1, ss1, 1).start(add=True)
                    _g(tab, rb0, gs0, 2).wait()
                    _s(rb1, ss1, 1).wait()
                    _g(tab, rb1, gs1, 3).start()
                    _s(rb0, ss0, 2).start(add=True)
                    _g(tab, rb1, gs1, 3).wait()
                    _s(rb0, ss0, 2).wait()
                    _s(rb1, ss1, 3).start(add=True)
                    _s(rb1, ss1, 3).wait()
            else:
                @pl.loop(0, ng)
                def _(g):
                    pltpu.sync_copy(sds.at[c, r, s, g], sdq)
                    _s(rb0, ss0, 0).start(add=True)
                    _s(rb0, ss1, 1).start(add=True)
                    _s(rb0, ss0, 0).wait()
                    _s(rb0, ss0, 2).start(add=True)
                    _s(rb0, ss1, 1).wait()
                    _s(rb0, ss1, 3).start(add=True)
                    _s(rb0, ss0, 2).wait()
                    _s(rb0, ss1, 3).wait()

            plsc.subcore_barrier()
            pltpu.sync_copy(acc.at[pl.ds(s * zr, zr)],
                            sums_o.at[c, r, pl.ds(s * zr, zr)])
            if r < n_rel - 1:
                pltpu.sync_copy(zf.at[pl.ds(s * zr, zr)],
                                acc.at[pl.ds(s * zr, zr)])
            plsc.subcore_barrier()

    return pl.kernel(body, out_type=out, mesh=mesh, scratch_types=scratch)


def _tc_combine0(n, bn, d, h):
    grid = (n // bn,)

    def body(S, C, xh, xc, xg, Wl, bl, Wr, oh, oc, og):
        m = (S[0] + S[1]) / jnp.maximum(C[0] + C[1], 1.0)
        og[...] = jax.nn.relu(_dot(m[0], Wl[0]) + bl[0] + _dot(xg[...], Wr[0]))
        oh[...] = jax.nn.relu(_dot(m[1], Wl[1]) + bl[1] + _dot(xh[...], Wr[1]))
        oc[...] = jax.nn.relu(
            _dot(m[2], Wl[2]) + _dot(m[3], Wl[3]) + _dot(m[4], Wl[4])
            + (bl[2] + bl[3] + bl[4])
            + _dot(xc[...], Wr[2] + Wr[3] + Wr[4]))

    return pl.pallas_call(
        body,
        grid=grid,
        in_specs=[
            pl.BlockSpec((2, 5, bn, d), lambda i: (0, 0, i, 0)),
            pl.BlockSpec((2, 5, bn, 1), lambda i: (0, 0, i, 0)),
            pl.BlockSpec((bn, d), lambda i: (i, 0)),
            pl.BlockSpec((bn, d), lambda i: (i, 0)),
            pl.BlockSpec((bn, d), lambda i: (i, 0)),
            pl.BlockSpec((5, d, h), lambda i: (0, 0, 0)),
            pl.BlockSpec((5, h), lambda i: (0, 0)),
            pl.BlockSpec((5, d, h), lambda i: (0, 0, 0)),
        ],
        out_specs=[
            pl.BlockSpec((bn, h), lambda i: (i, 0)),
            pl.BlockSpec((bn, h), lambda i: (i, 0)),
            pl.BlockSpec((bn, h), lambda i: (i, 0)),
        ],
        out_shape=[jax.ShapeDtypeStruct((n, h), jnp.float32)] * 3,
    )


def _tc_final(n, bn, h, h2, out_d):
    grid = (n // bn,)
    steps = n // bn

    def body(S, C, xc, Wl, bls, Wrs, b2d, f1w, f1b, f2w, f2b, out, accP, accC):
        i = pl.program_id(0)

        @pl.when(i == 0)
        def _():
            accP[...] = jnp.zeros_like(accP)
            accC[...] = jnp.zeros_like(accC)

        m = S[...] / jnp.maximum(C[0] + C[1], 1.0)
        o = jax.nn.relu(
            _dot(m[0], Wl[0]) + _dot(m[1], Wl[1]) + _dot(m[2], Wl[2])
            + bls[...] + _dot(xc[...], Wrs[...]))
        mask = (lax.broadcasted_iota(jnp.int32, (_G, bn), 0)
                == b2d[...].reshape(1, bn)).astype(jnp.float32)
        accP[...] += _dot(mask, o)
        accC[...] += jnp.broadcast_to(
            jnp.sum(mask, axis=1, keepdims=True), accC.shape)

        @pl.when(i == steps - 1)
        def _():
            pooled = accP[...] / jnp.maximum(accC[...], 1.0)
            hh = jax.nn.relu(_dot(pooled, f1w[...]) + f1b[...])
            out[...] = _dot(hh, f2w[...]) + f2b[...]

    return pl.pallas_call(
        body,
        grid=grid,
        in_specs=[
            pl.BlockSpec((3, bn, h), lambda i: (0, i, 0)),
            pl.BlockSpec((2, 3, bn, 1), lambda i: (0, 0, i, 0)),
            pl.BlockSpec((bn, h), lambda i: (i, 0)),
            pl.BlockSpec((3, h, h), lambda i: (0, 0, 0)),
            pl.BlockSpec((1, h), lambda i: (0, 0)),
            pl.BlockSpec((h, h), lambda i: (0, 0)),
            pl.BlockSpec((1, 1, bn), lambda i: (i, 0, 0)),
            pl.BlockSpec((h, h2), lambda i: (0, 0)),
            pl.BlockSpec((1, h2), lambda i: (0, 0)),
            pl.BlockSpec((h2, out_d), lambda i: (0, 0)),
            pl.BlockSpec((1, out_d), lambda i: (0, 0)),
        ],
        out_specs=pl.BlockSpec((_G, out_d), lambda i: (0, 0)),
        out_shape=jax.ShapeDtypeStruct((_G, out_d), jnp.float32),
        scratch_shapes=[
            pltpu.VMEM((_G, h), jnp.float32),
            pltpu.VMEM((_G, h), jnp.float32),
        ],
    )


def _split_flat(x, fc):
    return jnp.concatenate([x[:, :fc], x[:, fc:]], axis=0)


def kernel(x_hru, x_channel, x_gw_cell, ei_hru_gw, ei_ch_hru, ei_gw_ch,
           ei_hru_ch, ei_ch_ch, batch, Wl0, bl0, Wr0, Wl1, bl1, Wr1,
           fc1_w, fc1_b, fc2_w, fc2_b):
    n, d = x_hru.shape
    h = Wl0.shape[2]
    h2 = fc1_w.shape[1]
    out_d = fc2_w.shape[1]
    eis = [ei_hru_gw, ei_ch_hru, ei_gw_ch, ei_hru_ch, ei_ch_ch]
    e = eis[0].shape[1]

    np1 = -(-(n + 1) // 128) * 128

    chunk0 = _NC * _NS * _CH
    k0 = -(-(-(-e // chunk0)) // _GI) * _GI
    pad0 = k0 * chunk0 - e
    chunk1 = _NS * _CH
    k1 = -(-(-(-e // chunk1)) // _GI) * _GI
    pad1 = k1 * chunk1 - e
    psrc0 = jnp.arange(pad0, dtype=jnp.int32) % n
    pdst0 = n + jnp.arange(pad0, dtype=jnp.int32) % (np1 - n)
    psrc1 = jnp.arange(pad1, dtype=jnp.int32) % n
    pdst1 = n + jnp.arange(pad1, dtype=jnp.int32) % (np1 - n)
    src0_l, dst0_l, src1_l, dst1_l = [], [], [], []
    for ei in eis:
        s_ = ei[0].astype(jnp.int32)
        d_ = ei[1].astype(jnp.int32)
        src0_l.append(jnp.concatenate(
            [s_, psrc0]).reshape(_NC, _NS, k0, _CH))
        dst0_l.append(jnp.concatenate(
            [d_, pdst0]).reshape(_NC, _NS, k0, _CH))
        src1_l.append(jnp.concatenate(
            [s_, psrc1]).reshape(_NS, k1, _CH))
        dst1_l.append(jnp.concatenate(
            [d_, pdst1]).reshape(_NS, k1, _CH))
    srcs0 = jnp.stack(src0_l, axis=1)
    dsts0 = jnp.stack(dst0_l, axis=1)
    srcs0 = jnp.concatenate([srcs0, jnp.zeros_like(srcs0)], axis=1)
    dsts0 = jnp.concatenate([dsts0, dsts0], axis=1)
    sds0 = jnp.stack([srcs0, dsts0], axis=4).reshape(
        _NC, 10, _NS, k0 // _GI, _GI, 2, _CH)
    src1 = jnp.stack(src1_l[2:])
    dst1 = jnp.stack(dst1_l[2:])
    srcs1 = jnp.stack([src1, src1 + n])
    dsts1 = jnp.stack([dst1, dst1])
    sds1 = jnp.stack([srcs1, dsts1], axis=4).reshape(
        _NC, 3, _NS, k1 // _GI, _GI, 2, _CH)

    fc1 = h // 2
    zf0 = jnp.zeros((np1, d), jnp.float32)
    zf1 = jnp.zeros((np1, fc1), jnp.float32)
    ones0 = jnp.ones((_CH, d), jnp.float32)

    seg0 = _sc_segsum(10, (0, 1, 2, 0, 1, -1, -1, -1, -1, -1), 3, k0, np1, d)
    sums0 = seg0(x_hru, x_channel, x_gw_cell, sds0, zf0, ones0)
    S0 = sums0[:, :5, :n]
    C0 = sums0[:, 5:, :n, 0:1]

    bn = 1000
    xh1, xc1, xg1 = _tc_combine0(n, bn, d, h)(
        S0, C0, x_hru, x_channel, x_gw_cell, Wl0, bl0, Wr0)

    seg1 = _sc_segsum(3, (0, 1, 2), 3, k1, np1, fc1)
    sums1 = seg1(_split_flat(xg1, fc1), _split_flat(xh1, fc1),
                 _split_flat(xc1, fc1), sds1, zf1)
    S1 = jnp.concatenate([sums1[0], sums1[1]], axis=-1)[:, :n]
    C1 = C0[:, 2:5]

    out = _tc_final(n, bn, h, h2, out_d)(
        S1, C1, xc1, Wl1[2:5],
        (bl1[2] + bl1[3] + bl1[4]).reshape(1, h),
        Wr1[2] + Wr1[3] + Wr1[4],
        batch.astype(jnp.int32).reshape(n // bn, 1, bn),
        fc1_w, fc1_b.reshape(1, h2), fc2_w, fc2_b.reshape(1, out_d))
    return out


# speedup vs baseline: 2.6737x; 1.0596x over previous
import functools

import jax
import jax.numpy as jnp
from jax import lax
from jax.experimental import pallas as pl
from jax.experimental.pallas import tpu as pltpu
from jax.experimental.pallas import tpu_sc as plsc

_NC = 2
_NS = 16
_CH = 128
_G = 16

_HIGH = jax.lax.Precision.HIGHEST


def _dot(a, b):
    return jax.lax.dot_general(a, b, (((1,), (0,)), ((), ())),
                               precision=_HIGH,
                               preferred_element_type=jnp.float32)


_GI = 8


def _sc_segsum(n_rel, rel_tab, n_tabs, k, np1, fc):
    mesh = plsc.VectorSubcoreMesh(core_axis_name="c", subcore_axis_name="s",
                                  num_cores=_NC, num_subcores=_NS)
    out = jax.ShapeDtypeStruct((_NC, n_rel, np1, fc), jnp.float32)
    zr = np1 // _NS
    assert k % _GI == 0
    ng = k // _GI
    scratch = [
        pltpu.VMEM_SHARED((np1, fc), jnp.float32),
        pltpu.VMEM((_GI, 2, _CH), jnp.int32),
        pltpu.VMEM((2, _CH, fc), jnp.float32),
        pltpu.SemaphoreType.DMA,
        pltpu.SemaphoreType.DMA,
        pltpu.SemaphoreType.DMA,
        pltpu.SemaphoreType.DMA,
    ]

    first_cnt = min((i for i, t in enumerate(rel_tab) if t < 0),
                    default=None)

    def body(*refs):
        tabs = refs[:n_tabs]
        if first_cnt is not None:
            sds, zf, ones, sums_o = refs[n_tabs:n_tabs + 4]
            sc0 = n_tabs + 4
        else:
            sds, zf, sums_o = refs[n_tabs:n_tabs + 3]
            sc0 = n_tabs + 3
        acc, sdq, rb, gs0, gs1, ss0, ss1 = refs[sc0:]
        c = lax.axis_index("c")
        s = lax.axis_index("s")
        rb0, rb1 = rb.at[0], rb.at[1]

        def _g(tab, rbx, sem, i):
            return pltpu.make_async_copy(tab.at[sdq.at[i, 0]], rbx, sem)

        def _s(rbx, sem, i):
            return pltpu.make_async_copy(rbx, acc.at[sdq.at[i, 1]], sem)

        pltpu.sync_copy(zf.at[pl.ds(s * zr, zr)], acc.at[pl.ds(s * zr, zr)])
        plsc.subcore_barrier()
        for r in range(n_rel):
            if r == first_cnt:
                pltpu.sync_copy(ones, rb0)
            rbs = (rb0, rb1)
            gss = (gs0, gs1)
            sss = (ss0, ss1)
            if rel_tab[r] >= 0:
                tab = tabs[rel_tab[r]]

                @pl.loop(0, ng)
                def _(g):
                    pltpu.sync_copy(sds.at[c, r, s, g], sdq)
                    _g(tab, rb0, gs0, 0).start()
                    _g(tab, rb1, gs1, 1).start()
                    _g(tab, rb0, gs0, 0).wait()
                    _s(rb0, ss0, 0).start(add=True)
                    for b in range(1, _GI):
                        i, p = b & 1, (b - 1) & 1
                        _g(tab, rbs[i], gss[i], b).wait()
                        _s(rbs[p], sss[p], b - 1).wait()
                        if b + 1 < _GI:
                            _g(tab, rbs[p], gss[p], b + 1).start()
                        _s(rbs[i], sss[i], b).start(add=True)
                    _s(rbs[(_GI - 1) & 1], sss[(_GI - 1) & 1],
                       _GI - 1).wait()
            else:
                csem = (ss0, ss1, gs0, gs1)

                @pl.loop(0, ng)
                def _(g):
                    pltpu.sync_copy(sds.at[c, r, s, g], sdq)
                    for b in range(_GI):
                        if b >= 4:
                            _s(rb0, csem[b & 3], b - 4).wait()
                        _s(rb0, csem[b & 3], b).start(add=True)
                    for b in range(_GI - 4, _GI):
                        _s(rb0, csem[b & 3], b).wait()

            plsc.subcore_barrier()
            pltpu.sync_copy(acc.at[pl.ds(s * zr, zr)],
                            sums_o.at[c, r, pl.ds(s * zr, zr)])
            if r < n_rel - 1:
                pltpu.sync_copy(zf.at[pl.ds(s * zr, zr)],
                                acc.at[pl.ds(s * zr, zr)])
            plsc.subcore_barrier()

    return pl.kernel(body, out_type=out, mesh=mesh, scratch_types=scratch)


def _tc_combine0(n, bn, d, h):
    grid = (n // bn,)

    def body(S, C, xh, xc, xg, Wl, bl, Wr, oh, oc, og):
        m = (S[0] + S[1]) / jnp.maximum(C[0] + C[1], 1.0)
        og[...] = jax.nn.relu(_dot(m[0], Wl[0]) + bl[0] + _dot(xg[...], Wr[0]))
        oh[...] = jax.nn.relu(_dot(m[1], Wl[1]) + bl[1] + _dot(xh[...], Wr[1]))
        oc[...] = jax.nn.relu(
            _dot(m[2], Wl[2]) + _dot(m[3], Wl[3]) + _dot(m[4], Wl[4])
            + (bl[2] + bl[3] + bl[4])
            + _dot(xc[...], Wr[2] + Wr[3] + Wr[4]))

    return pl.pallas_call(
        body,
        grid=grid,
        in_specs=[
            pl.BlockSpec((2, 5, bn, d), lambda i: (0, 0, i, 0)),
            pl.BlockSpec((2, 5, bn, 1), lambda i: (0, 0, i, 0)),
            pl.BlockSpec((bn, d), lambda i: (i, 0)),
            pl.BlockSpec((bn, d), lambda i: (i, 0)),
            pl.BlockSpec((bn, d), lambda i: (i, 0)),
            pl.BlockSpec((5, d, h), lambda i: (0, 0, 0)),
            pl.BlockSpec((5, h), lambda i: (0, 0)),
            pl.BlockSpec((5, d, h), lambda i: (0, 0, 0)),
        ],
        out_specs=[
            pl.BlockSpec((bn, h), lambda i: (i, 0)),
            pl.BlockSpec((bn, h), lambda i: (i, 0)),
            pl.BlockSpec((bn, h), lambda i: (i, 0)),
        ],
        out_shape=[jax.ShapeDtypeStruct((n, h), jnp.float32)] * 3,
    )


def _tc_final(n, bn, h, h2, out_d):
    grid = (n // bn,)
    steps = n // bn

    def body(S, C, xc, Wl, bls, Wrs, b2d, f1w, f1b, f2w, f2b, out, accP, accC):
        i = pl.program_id(0)

        @pl.when(i == 0)
        def _():
            accP[...] = jnp.zeros_like(accP)
            accC[...] = jnp.zeros_like(accC)

        m = S[...] / jnp.maximum(C[0] + C[1], 1.0)
        o = jax.nn.relu(
            _dot(m[0], Wl[0]) + _dot(m[1], Wl[1]) + _dot(m[2], Wl[2])
            + bls[...] + _dot(xc[...], Wrs[...]))
        mask = (lax.broadcasted_iota(jnp.int32, (_G, bn), 0)
                == b2d[...].reshape(1, bn)).astype(jnp.float32)
        accP[...] += _dot(mask, o)
        accC[...] += jnp.broadcast_to(
            jnp.sum(mask, axis=1, keepdims=True), accC.shape)

        @pl.when(i == steps - 1)
        def _():
            pooled = accP[...] / jnp.maximum(accC[...], 1.0)
            hh = jax.nn.relu(_dot(pooled, f1w[...]) + f1b[...])
            out[...] = _dot(hh, f2w[...]) + f2b[...]

    return pl.pallas_call(
        body,
        grid=grid,
        in_specs=[
            pl.BlockSpec((3, bn, h), lambda i: (0, i, 0)),
            pl.BlockSpec((2, 3, bn, 1), lambda i: (0, 0, i, 0)),
            pl.BlockSpec((bn, h), lambda i: (i, 0)),
            pl.BlockSpec((3, h, h), lambda i: (0, 0, 0)),
            pl.BlockSpec((1, h), lambda i: (0, 0)),
            pl.BlockSpec((h, h), lambda i: (0, 0)),
            pl.BlockSpec((1, 1, bn), lambda i: (i, 0, 0)),
            pl.BlockSpec((h, h2), lambda i: (0, 0)),
            pl.BlockSpec((1, h2), lambda i: (0, 0)),
            pl.BlockSpec((h2, out_d), lambda i: (0, 0)),
            pl.BlockSpec((1, out_d), lambda i: (0, 0)),
        ],
        out_specs=pl.BlockSpec((_G, out_d), lambda i: (0, 0)),
        out_shape=jax.ShapeDtypeStruct((_G, out_d), jnp.float32),
        scratch_shapes=[
            pltpu.VMEM((_G, h), jnp.float32),
            pltpu.VMEM((_G, h), jnp.float32),
        ],
    )


def _split_flat(x, fc):
    return jnp.concatenate([x[:, :fc], x[:, fc:]], axis=0)


def kernel(x_hru, x_channel, x_gw_cell, ei_hru_gw, ei_ch_hru, ei_gw_ch,
           ei_hru_ch, ei_ch_ch, batch, Wl0, bl0, Wr0, Wl1, bl1, Wr1,
           fc1_w, fc1_b, fc2_w, fc2_b):
    n, d = x_hru.shape
    h = Wl0.shape[2]
    h2 = fc1_w.shape[1]
    out_d = fc2_w.shape[1]
    eis = [ei_hru_gw, ei_ch_hru, ei_gw_ch, ei_hru_ch, ei_ch_ch]
    e = eis[0].shape[1]

    np1 = -(-(n + 1) // 128) * 128

    chunk0 = _NC * _NS * _CH
    k0 = -(-(-(-e // chunk0)) // _GI) * _GI
    pad0 = k0 * chunk0 - e
    chunk1 = _NS * _CH
    k1 = -(-(-(-e // chunk1)) // _GI) * _GI
    pad1 = k1 * chunk1 - e
    psrc0 = jnp.arange(pad0, dtype=jnp.int32) % n
    pdst0 = n + jnp.arange(pad0, dtype=jnp.int32) % (np1 - n)
    psrc1 = jnp.arange(pad1, dtype=jnp.int32) % n
    pdst1 = n + jnp.arange(pad1, dtype=jnp.int32) % (np1 - n)
    src0_l, dst0_l, src1_l, dst1_l = [], [], [], []
    for ei in eis:
        s_ = ei[0].astype(jnp.int32)
        d_ = ei[1].astype(jnp.int32)
        src0_l.append(jnp.concatenate(
            [s_, psrc0]).reshape(_NC, _NS, k0, _CH))
        dst0_l.append(jnp.concatenate(
            [d_, pdst0]).reshape(_NC, _NS, k0, _CH))
        src1_l.append(jnp.concatenate(
            [s_, psrc1]).reshape(_NS, k1, _CH))
        dst1_l.append(jnp.concatenate(
            [d_, pdst1]).reshape(_NS, k1, _CH))
    srcs0 = jnp.stack(src0_l, axis=1)
    dsts0 = jnp.stack(dst0_l, axis=1)
    srcs0 = jnp.concatenate([srcs0, jnp.zeros_like(srcs0)], axis=1)
    dsts0 = jnp.concatenate([dsts0, dsts0], axis=1)
    sds0 = jnp.stack([srcs0, dsts0], axis=4).reshape(
        _NC, 10, _NS, k0 // _GI, _GI, 2, _CH)
    src1 = jnp.stack(src1_l[2:])
    dst1 = jnp.stack(dst1_l[2:])
    srcs1 = jnp.stack([src1, src1 + n])
    dsts1 = jnp.stack([dst1, dst1])
    sds1 = jnp.stack([srcs1, dsts1], axis=4).reshape(
        _NC, 3, _NS, k1 // _GI, _GI, 2, _CH)

    fc1 = h // 2
    zf0 = jnp.zeros((np1, d), jnp.float32)
    zf1 = jnp.zeros((np1, fc1), jnp.float32)
    ones0 = jnp.ones((_CH, d), jnp.float32)

    seg0 = _sc_segsum(10, (0, 1, 2, 0, 1, -1, -1, -1, -1, -1), 3, k0, np1, d)
    sums0 = seg0(x_hru, x_channel, x_gw_cell, sds0, zf0, ones0)
    S0 = sums0[:, :5, :n]
    C0 = sums0[:, 5:, :n, 0:1]

    bn = 1000
    xh1, xc1, xg1 = _tc_combine0(n, bn, d, h)(
        S0, C0, x_hru, x_channel, x_gw_cell, Wl0, bl0, Wr0)

    seg1 = _sc_segsum(3, (0, 1, 2), 3, k1, np1, fc1)
    sums1 = seg1(_split_flat(xg1, fc1), _split_flat(xh1, fc1),
                 _split_flat(xc1, fc1), sds1, zf1)
    S1 = jnp.concatenate([sums1[0], sums1[1]], axis=-1)[:, :n]
    C1 = C0[:, 2:5]

    out = _tc_final(n, bn, h, h2, out_d)(
        S1, C1, xc1, Wl1[2:5],
        (bl1[2] + bl1[3] + bl1[4]).reshape(1, h),
        Wr1[2] + Wr1[3] + Wr1[4],
        batch.astype(jnp.int32).reshape(n // bn, 1, bn),
        fc1_w, fc1_b.reshape(1, h2), fc2_w, fc2_b.reshape(1, out_d))
    return out


# speedup vs baseline: 2.7762x; 1.0383x over previous
import functools

import jax
import jax.numpy as jnp
from jax import lax
from jax.experimental import pallas as pl
from jax.experimental.pallas import tpu as pltpu
from jax.experimental.pallas import tpu_sc as plsc

_NC = 2
_NS = 16
_CH = 128
_G = 16

_HIGH = jax.lax.Precision.HIGHEST


def _dot(a, b):
    return jax.lax.dot_general(a, b, (((1,), (0,)), ((), ())),
                               precision=_HIGH,
                               preferred_element_type=jnp.float32)


_GI = 16


def _sc_segsum(n_rel, rel_tab, n_tabs, k, np1, fc):
    mesh = plsc.VectorSubcoreMesh(core_axis_name="c", subcore_axis_name="s",
                                  num_cores=_NC, num_subcores=_NS)
    out = jax.ShapeDtypeStruct((_NC, n_rel, np1, fc), jnp.float32)
    zr = np1 // _NS
    assert k % _GI == 0
    ng = k // _GI
    scratch = [
        pltpu.VMEM_SHARED((np1, fc), jnp.float32),
        pltpu.VMEM((_GI, 2, _CH), jnp.int32),
        pltpu.VMEM((2, _CH, fc), jnp.float32),
        pltpu.SemaphoreType.DMA,
        pltpu.SemaphoreType.DMA,
        pltpu.SemaphoreType.DMA,
        pltpu.SemaphoreType.DMA,
    ]

    first_cnt = min((i for i, t in enumerate(rel_tab) if t < 0),
                    default=None)

    def body(*refs):
        tabs = refs[:n_tabs]
        if first_cnt is not None:
            sds, zf, ones, sums_o = refs[n_tabs:n_tabs + 4]
            sc0 = n_tabs + 4
        else:
            sds, zf, sums_o = refs[n_tabs:n_tabs + 3]
            sc0 = n_tabs + 3
        acc, sdq, rb, gs0, gs1, ss0, ss1 = refs[sc0:]
        c = lax.axis_index("c")
        s = lax.axis_index("s")
        rb0, rb1 = rb.at[0], rb.at[1]

        def _g(tab, rbx, sem, i):
            return pltpu.make_async_copy(tab.at[sdq.at[i, 0]], rbx, sem)

        def _s(rbx, sem, i):
            return pltpu.make_async_copy(rbx, acc.at[sdq.at[i, 1]], sem)

        pltpu.sync_copy(zf.at[pl.ds(s * zr, zr)], acc.at[pl.ds(s * zr, zr)])
        plsc.subcore_barrier()
        for r in range(n_rel):
            if r == first_cnt:
                pltpu.sync_copy(ones, rb0)
            rbs = (rb0, rb1)
            gss = (gs0, gs1)
            sss = (ss0, ss1)
            if rel_tab[r] >= 0:
                tab = tabs[rel_tab[r]]

                @pl.loop(0, ng)
                def _(g):
                    pltpu.sync_copy(sds.at[c, r, s, g], sdq)
                    _g(tab, rb0, gs0, 0).start()
                    _g(tab, rb1, gs1, 1).start()
                    _g(tab, rb0, gs0, 0).wait()
                    _s(rb0, ss0, 0).start(add=True)
                    for b in range(1, _GI):
                        i, p = b & 1, (b - 1) & 1
                        _g(tab, rbs[i], gss[i], b).wait()
                        _s(rbs[p], sss[p], b - 1).wait()
                        if b + 1 < _GI:
                            _g(tab, rbs[p], gss[p], b + 1).start()
                        _s(rbs[i], sss[i], b).start(add=True)
                    _s(rbs[(_GI - 1) & 1], sss[(_GI - 1) & 1],
                       _GI - 1).wait()
            else:
                csem = (ss0, ss1, gs0, gs1)

                @pl.loop(0, ng)
                def _(g):
                    pltpu.sync_copy(sds.at[c, r, s, g], sdq)
                    for b in range(_GI):
                        if b >= 4:
                            _s(rb0, csem[b & 3], b - 4).wait()
                        _s(rb0, csem[b & 3], b).start(add=True)
                    for b in range(_GI - 4, _GI):
                        _s(rb0, csem[b & 3], b).wait()

            plsc.subcore_barrier()
            pltpu.sync_copy(acc.at[pl.ds(s * zr, zr)],
                            sums_o.at[c, r, pl.ds(s * zr, zr)])
            if r < n_rel - 1:
                pltpu.sync_copy(zf.at[pl.ds(s * zr, zr)],
                                acc.at[pl.ds(s * zr, zr)])
            plsc.subcore_barrier()

    return pl.kernel(body, out_type=out, mesh=mesh, scratch_types=scratch)


def _tc_combine0(n, bn, d, h):
    grid = (n // bn,)

    def body(S, C, xh, xc, xg, Wl, bl, Wr, oh, oc, og):
        m = (S[0] + S[1]) / jnp.maximum(C[0] + C[1], 1.0)
        og[...] = jax.nn.relu(_dot(m[0], Wl[0]) + bl[0] + _dot(xg[...], Wr[0]))
        oh[...] = jax.nn.relu(_dot(m[1], Wl[1]) + bl[1] + _dot(xh[...], Wr[1]))
        oc[...] = jax.nn.relu(
            _dot(m[2], Wl[2]) + _dot(m[3], Wl[3]) + _dot(m[4], Wl[4])
            + (bl[2] + bl[3] + bl[4])
            + _dot(xc[...], Wr[2] + Wr[3] + Wr[4]))

    return pl.pallas_call(
        body,
        grid=grid,
        in_specs=[
            pl.BlockSpec((2, 5, bn, d), lambda i: (0, 0, i, 0)),
            pl.BlockSpec((2, 5, bn, 1), lambda i: (0, 0, i, 0)),
            pl.BlockSpec((bn, d), lambda i: (i, 0)),
            pl.BlockSpec((bn, d), lambda i: (i, 0)),
            pl.BlockSpec((bn, d), lambda i: (i, 0)),
            pl.BlockSpec((5, d, h), lambda i: (0, 0, 0)),
            pl.BlockSpec((5, h), lambda i: (0, 0)),
            pl.BlockSpec((5, d, h), lambda i: (0, 0, 0)),
        ],
        out_specs=[
            pl.BlockSpec((bn, h), lambda i: (i, 0)),
            pl.BlockSpec((bn, h), lambda i: (i, 0)),
            pl.BlockSpec((bn, h), lambda i: (i, 0)),
        ],
        out_shape=[jax.ShapeDtypeStruct((n, h), jnp.float32)] * 3,
    )


def _tc_final(n, bn, h, h2, out_d):
    grid = (n // bn,)
    steps = n // bn

    def body(S, C, xc, Wl, bls, Wrs, b2d, f1w, f1b, f2w, f2b, out, accP, accC):
        i = pl.program_id(0)

        @pl.when(i == 0)
        def _():
            accP[...] = jnp.zeros_like(accP)
            accC[...] = jnp.zeros_like(accC)

        m = S[...] / jnp.maximum(C[0] + C[1], 1.0)
        o = jax.nn.relu(
            _dot(m[0], Wl[0]) + _dot(m[1], Wl[1]) + _dot(m[2], Wl[2])
            + bls[...] + _dot(xc[...], Wrs[...]))
        mask = (lax.broadcasted_iota(jnp.int32, (_G, bn), 0)
                == b2d[...].reshape(1, bn)).astype(jnp.float32)
        accP[...] += _dot(mask, o)
        accC[...] += jnp.broadcast_to(
            jnp.sum(mask, axis=1, keepdims=True), accC.shape)

        @pl.when(i == steps - 1)
        def _():
            pooled = accP[...] / jnp.maximum(accC[...], 1.0)
            hh = jax.nn.relu(_dot(pooled, f1w[...]) + f1b[...])
            out[...] = _dot(hh, f2w[...]) + f2b[...]

    return pl.pallas_call(
        body,
        grid=grid,
        in_specs=[
            pl.BlockSpec((3, bn, h), lambda i: (0, i, 0)),
            pl.BlockSpec((2, 3, bn, 1), lambda i: (0, 0, i, 0)),
            pl.BlockSpec((bn, h), lambda i: (i, 0)),
            pl.BlockSpec((3, h, h), lambda i: (0, 0, 0)),
            pl.BlockSpec((1, h), lambda i: (0, 0)),
            pl.BlockSpec((h, h), lambda i: (0, 0)),
            pl.BlockSpec((1, 1, bn), lambda i: (i, 0, 0)),
            pl.BlockSpec((h, h2), lambda i: (0, 0)),
            pl.BlockSpec((1, h2), lambda i: (0, 0)),
            pl.BlockSpec((h2, out_d), lambda i: (0, 0)),
            pl.BlockSpec((1, out_d), lambda i: (0, 0)),
        ],
        out_specs=pl.BlockSpec((_G, out_d), lambda i: (0, 0)),
        out_shape=jax.ShapeDtypeStruct((_G, out_d), jnp.float32),
        scratch_shapes=[
            pltpu.VMEM((_G, h), jnp.float32),
            pltpu.VMEM((_G, h), jnp.float32),
        ],
    )


def _split_flat(x, fc):
    return jnp.concatenate([x[:, :fc], x[:, fc:]], axis=0)


def kernel(x_hru, x_channel, x_gw_cell, ei_hru_gw, ei_ch_hru, ei_gw_ch,
           ei_hru_ch, ei_ch_ch, batch, Wl0, bl0, Wr0, Wl1, bl1, Wr1,
           fc1_w, fc1_b, fc2_w, fc2_b):
    n, d = x_hru.shape
    h = Wl0.shape[2]
    h2 = fc1_w.shape[1]
    out_d = fc2_w.shape[1]
    eis = [ei_hru_gw, ei_ch_hru, ei_gw_ch, ei_hru_ch, ei_ch_ch]
    e = eis[0].shape[1]

    np1 = -(-(n + 1) // 128) * 128

    chunk0 = _NC * _NS * _CH
    k0 = -(-(-(-e // chunk0)) // _GI) * _GI
    pad0 = k0 * chunk0 - e
    chunk1 = _NS * _CH
    k1 = -(-(-(-e // chunk1)) // _GI) * _GI
    pad1 = k1 * chunk1 - e
    psrc0 = jnp.arange(pad0, dtype=jnp.int32) % n
    pdst0 = n + jnp.arange(pad0, dtype=jnp.int32) % (np1 - n)
    psrc1 = jnp.arange(pad1, dtype=jnp.int32) % n
    pdst1 = n + jnp.arange(pad1, dtype=jnp.int32) % (np1 - n)
    src0_l, dst0_l, src1_l, dst1_l = [], [], [], []
    for ei in eis:
        s_ = ei[0].astype(jnp.int32)
        d_ = ei[1].astype(jnp.int32)
        src0_l.append(jnp.concatenate(
            [s_, psrc0]).reshape(_NC, _NS, k0, _CH))
        dst0_l.append(jnp.concatenate(
            [d_, pdst0]).reshape(_NC, _NS, k0, _CH))
        src1_l.append(jnp.concatenate(
            [s_, psrc1]).reshape(_NS, k1, _CH))
        dst1_l.append(jnp.concatenate(
            [d_, pdst1]).reshape(_NS, k1, _CH))
    srcs0 = jnp.stack(src0_l, axis=1)
    dsts0 = jnp.stack(dst0_l, axis=1)
    srcs0 = jnp.concatenate([srcs0, jnp.zeros_like(srcs0)], axis=1)
    dsts0 = jnp.concatenate([dsts0, dsts0], axis=1)
    sds0 = jnp.stack([srcs0, dsts0], axis=4).reshape(
        _NC, 10, _NS, k0 // _GI, _GI, 2, _CH)
    src1 = jnp.stack(src1_l[2:])
    dst1 = jnp.stack(dst1_l[2:])
    srcs1 = jnp.stack([src1, src1 + n])
    dsts1 = jnp.stack([dst1, dst1])
    sds1 = jnp.stack([srcs1, dsts1], axis=4).reshape(
        _NC, 3, _NS, k1 // _GI, _GI, 2, _CH)

    fc1 = h // 2
    zf0 = jnp.zeros((np1, d), jnp.float32)
    zf1 = jnp.zeros((np1, fc1), jnp.float32)
    ones0 = jnp.ones((_CH, d), jnp.float32)

    seg0 = _sc_segsum(10, (0, 1, 2, 0, 1, -1, -1, -1, -1, -1), 3, k0, np1, d)
    sums0 = seg0(x_hru, x_channel, x_gw_cell, sds0, zf0, ones0)
    S0 = sums0[:, :5, :n]
    C0 = sums0[:, 5:, :n, 0:1]

    bn = 1000
    xh1, xc1, xg1 = _tc_combine0(n, bn, d, h)(
        S0, C0, x_hru, x_channel, x_gw_cell, Wl0, bl0, Wr0)

    seg1 = _sc_segsum(3, (0, 1, 2), 3, k1, np1, fc1)
    sums1 = seg1(_split_flat(xg1, fc1), _split_flat(xh1, fc1),
                 _split_flat(xc1, fc1), sds1, zf1)
    S1 = jnp.concatenate([sums1[0], sums1[1]], axis=-1)[:, :n]
    C1 = C0[:, 2:5]

    out = _tc_final(n, bn, h, h2, out_d)(
        S1, C1, xc1, Wl1[2:5],
        (bl1[2] + bl1[3] + bl1[4]).reshape(1, h),
        Wr1[2] + Wr1[3] + Wr1[4],
        batch.astype(jnp.int32).reshape(n // bn, 1, bn),
        fc1_w, fc1_b.reshape(1, h2), fc2_w, fc2_b.reshape(1, out_d))
    return out
